# baseline (speedup 1.0000x reference)
import functools

import jax
import jax.numpy as jnp
from jax import lax
from jax.experimental import pallas as pl
from jax.experimental.pallas import tpu as pltpu
from jax.experimental.pallas import tpu_sc as plsc

N = 10000
E = 320000
D = 128
H = 128
B = 64

NC, NS = 2, 16
NW = NC * NS
EPW = E // NW
ECHUNK = 100
NECHUNK = EPW // ECHUNK
NPH = 5
CPP = NECHUNK // NPH
RING = 2
RCHUNK = 40
NRCHUNK = N // RCHUNK
RCPT = -(-NRCHUNK // NS)


def _sc_mesh():
    return plsc.VectorSubcoreMesh(core_axis_name="c", subcore_axis_name="s")


def _deg_body(dst_hbm, out_hbm, di_v, ones_v, zb_v, hist_sh):
    c = lax.axis_index("c")
    s = lax.axis_index("s")
    w = c * NS + s

    def fill(i, _):
        ones_v[i, :] = jnp.ones((16,), jnp.float32)
        return 0

    lax.fori_loop(0, ECHUNK, fill, 0)

    def zfill(i, _):
        zb_v[i, :] = jnp.zeros((16,), jnp.float32)
        return 0

    lax.fori_loop(0, RCHUNK, zfill, 0)
    pltpu.sync_copy(dst_hbm.at[w], di_v)

    def zstep(j, _):
        rc = s + NS * j

        @pl.when(rc < NRCHUNK)
        def _():
            pltpu.sync_copy(zb_v, hist_sh.at[pl.ds(rc * RCHUNK, RCHUNK)])
        return 0

    lax.fori_loop(0, RCPT, zstep, 0)
    plsc.subcore_barrier()

    def step(t, _):
        pltpu.sync_copy(ones_v, hist_sh.at[di_v.at[t]], add=True)
        return 0

    lax.fori_loop(0, NECHUNK, step, 0)
    plsc.subcore_barrier()

    def wstep(j, _):
        rc = s + NS * j

        @pl.when(rc < NRCHUNK)
        def _():
            pltpu.sync_copy(hist_sh.at[pl.ds(rc * RCHUNK, RCHUNK)],
                            out_hbm.at[c, pl.ds(rc * RCHUNK, RCHUNK)])
        return 0

    lax.fori_loop(0, RCPT, wstep, 0)


def _deg(dst3d):
    f = pl.kernel(
        _deg_body,
        out_type=jax.ShapeDtypeStruct((NC, N, 16), jnp.float32),
        mesh=_sc_mesh(),
        scratch_types=[
            pltpu.VMEM((NECHUNK, ECHUNK), jnp.int32),
            pltpu.VMEM((ECHUNK, 16), jnp.float32),
            pltpu.VMEM((RCHUNK, 16), jnp.float32),
            pltpu.VMEM_SHARED((N, 16), jnp.float32),
        ],
    )
    return f(dst3d)


def _agg_body(src_hbm, dst_hbm, y_hbm, out_hbm, si_v, di_v, rows,
              acc_sh, gsems):
    c = lax.axis_index("c")
    s = lax.axis_index("s")
    w = c * NS + s

    def zfill(i, _):
        for k in range(H // 16):
            rows[0][i, pl.ds(k * 16, 16)] = jnp.zeros((16,), jnp.float32)
        return 0

    lax.fori_loop(0, RCHUNK, zfill, 0)

    def zstep(j, _):
        rc = s + NS * j

        @pl.when(rc < NRCHUNK)
        def _():
            pltpu.sync_copy(rows[0].at[pl.ds(0, RCHUNK)],
                            acc_sh.at[pl.ds(rc * RCHUNK, RCHUNK)])
        return 0

    lax.fori_loop(0, RCPT, zstep, 0)
    plsc.subcore_barrier()

    for p in range(NPH):
        pltpu.sync_copy(src_hbm.at[w, p], si_v)
        pltpu.sync_copy(dst_hbm.at[w, p], di_v)
        for k in range(RING):
            pltpu.async_copy(y_hbm.at[si_v.at[k]], rows[k], gsems[k])

        def step(t, _):
            j0 = RING * t
            for k in range(RING):
                pltpu.make_async_copy(y_hbm.at[si_v.at[j0 + k]], rows[k],
                                      gsems[k]).wait()
                pltpu.sync_copy(rows[k], acc_sh.at[di_v.at[j0 + k]], add=True)

                @pl.when(t < CPP // RING - 1)
                def _():
                    pltpu.async_copy(y_hbm.at[si_v.at[j0 + RING + k]],
                                     rows[k], gsems[k])
            return 0

        lax.fori_loop(0, CPP // RING, step, 0)
    plsc.subcore_barrier()

    def wstep(j, _):
        rc = s + NS * j

        @pl.when(rc < NRCHUNK)
        def _():
            pltpu.sync_copy(acc_sh.at[pl.ds(rc * RCHUNK, RCHUNK)],
                            out_hbm.at[c, pl.ds(rc * RCHUNK, RCHUNK)])
        return 0

    lax.fori_loop(0, RCPT, wstep, 0)


def _agg_body_flat(src_hbm, dst_hbm, y_hbm, out_hbm, si_v, di_v,
                   r0, r1, acc_sh, g0, g1):
    return _agg_body(src_hbm, dst_hbm, y_hbm, out_hbm, si_v, di_v,
                     [r0, r1], acc_sh, [g0, g1])


def _agg(src3d, dst3d, y):
    f = pl.kernel(
        _agg_body_flat,
        out_type=jax.ShapeDtypeStruct((NC, N, H), jnp.float32),
        mesh=_sc_mesh(),
        scratch_types=(
            [pltpu.VMEM((CPP, ECHUNK), jnp.int32)] * 2
            + [pltpu.VMEM((ECHUNK, H), jnp.float32)] * RING
            + [pltpu.VMEM_SHARED((N, H), jnp.float32)]
            + [pltpu.SemaphoreType.DMA] * RING
        ),
    )
    return f(src3d, dst3d, y)


R = 1000
GRID = N // R


def _dinv(hp):
    deg = hp[0, :, 0] + hp[1, :, 0] + 1.0
    return lax.rsqrt(deg)


def _xw_body(hp_ref, x_ref, w_ref, y_ref):
    dinv = _dinv(hp_ref[...])
    y_ref[...] = dinv[:, None] * jnp.dot(x_ref[...], w_ref[...],
                                         preferred_element_type=jnp.float32)


def _xw(hp, x, w):
    return pl.pallas_call(
        _xw_body,
        grid=(GRID,),
        in_specs=[
            pl.BlockSpec((NC, R, 16), lambda i: (0, i, 0)),
            pl.BlockSpec((R, D), lambda i: (i, 0)),
            pl.BlockSpec((D, H), lambda i: (0, 0)),
        ],
        out_specs=pl.BlockSpec((R, H), lambda i: (i, 0)),
        out_shape=jax.ShapeDtypeStruct((N, H), jnp.float32),
    )(hp, x, w)


def _layer_body(hp_ref, ap_ref, y_ref, w_ref, b_ref, o_ref):
    dinv = _dinv(hp_ref[...])
    aa = ap_ref[...]
    a = aa[0] + aa[1] + y_ref[...]
    h = jnp.maximum(dinv[:, None] * a + b_ref[...], 0.0)
    o_ref[...] = dinv[:, None] * jnp.dot(h, w_ref[...],
                                         preferred_element_type=jnp.float32)


def _layer(hp, ap, y, w, b):
    return pl.pallas_call(
        _layer_body,
        grid=(GRID,),
        in_specs=[
            pl.BlockSpec((NC, R, 16), lambda i: (0, i, 0)),
            pl.BlockSpec((NC, R, H), lambda i: (0, i, 0)),
            pl.BlockSpec((R, H), lambda i: (i, 0)),
            pl.BlockSpec((H, H), lambda i: (0, 0)),
            pl.BlockSpec((1, H), lambda i: (0, 0)),
        ],
        out_specs=pl.BlockSpec((R, H), lambda i: (i, 0)),
        out_shape=jax.ShapeDtypeStruct((N, H), jnp.float32),
    )(hp, ap, y, w, b)


def _final_body(hp_ref, ap_ref, y_ref, b2_ref, batch_ref, l1w_ref, l1b_ref,
                l2w_ref, l2b_ref, o_ref, pooled_acc, cnt_acc):
    i = pl.program_id(0)
    dinv = _dinv(hp_ref[...])
    aa = ap_ref[...]
    a = aa[0] + aa[1] + y_ref[...]
    h = jnp.maximum(dinv[:, None] * a + b2_ref[...], 0.0)

    bid = batch_ref[...]
    lanes = lax.broadcasted_iota(jnp.int32, (R, B), 1)
    m = jnp.where(bid == lanes, 1.0, 0.0)
    p = lax.dot_general(m, h, (((0,), (0,)), ((), ())),
                        preferred_element_type=jnp.float32)
    cn = lax.dot_general(m, jnp.ones_like(h), (((0,), (0,)), ((), ())),
                         preferred_element_type=jnp.float32)

    @pl.when(i == 0)
    def _():
        pooled_acc[...] = jnp.zeros_like(pooled_acc)
        cnt_acc[...] = jnp.zeros_like(cnt_acc)

    pooled_acc[...] += p
    cnt_acc[...] += cn

    @pl.when(i == GRID - 1)
    def _():
        mean = pooled_acc[...] / jnp.maximum(cnt_acc[...], 1.0)
        z = jnp.maximum(jnp.dot(mean, l1w_ref[...],
                                preferred_element_type=jnp.float32)
                        + l1b_ref[...], 0.0)
        o_ref[...] = (jnp.dot(z, l2w_ref[...],
                              preferred_element_type=jnp.float32)
                      + l2b_ref[...])


def _final(hp, ap, y, b2, batch2d, l1w, l1b, l2w, l2b):
    return pl.pallas_call(
        _final_body,
        grid=(GRID,),
        in_specs=[
            pl.BlockSpec((NC, R, 16), lambda i: (0, i, 0)),
            pl.BlockSpec((NC, R, H), lambda i: (0, i, 0)),
            pl.BlockSpec((R, H), lambda i: (i, 0)),
            pl.BlockSpec((1, H), lambda i: (0, 0)),
            pl.BlockSpec((R, 1), lambda i: (i, 0)),
            pl.BlockSpec((H, B), lambda i: (0, 0)),
            pl.BlockSpec((1, B), lambda i: (0, 0)),
            pl.BlockSpec((B, 1), lambda i: (0, 0)),
            pl.BlockSpec((1, 1), lambda i: (0, 0)),
        ],
        out_specs=pl.BlockSpec((B, 1), lambda i: (0, 0)),
        out_shape=jax.ShapeDtypeStruct((B, 1), jnp.float32),
        scratch_shapes=[
            pltpu.VMEM((B, H), jnp.float32),
            pltpu.VMEM((B, H), jnp.float32),
        ],
        compiler_params=pltpu.CompilerParams(
            dimension_semantics=("arbitrary",)),
    )(hp, ap, y, b2, batch2d, l1w, l1b, l2w, l2b)


def kernel(x, edge_index, batch, W1, b1, W2, b2, l1W, l1b, l2W, l2b):
    src = edge_index[0].astype(jnp.int32).reshape(NW, NPH, CPP, ECHUNK)
    dst = edge_index[1].astype(jnp.int32).reshape(NW, NPH, CPP, ECHUNK)
    hp = _deg(dst.reshape(NW, NECHUNK, ECHUNK))
    y1 = _xw(hp, x, W1)
    ap1 = _agg(src, dst, y1)
    y2 = _layer(hp, ap1, y1, W2, b1.reshape(1, H))
    ap2 = _agg(src, dst, y2)
    out = _final(hp, ap2, y2, b2.reshape(1, H), batch.astype(jnp.int32).reshape(N, 1),
                 l1W, l1b.reshape(1, B), l2W, l2b.reshape(1, 1))
    return out[:, 0]


# speedup vs baseline: 1.0569x; 1.0569x over previous
import functools

import jax
import jax.numpy as jnp
from jax import lax
from jax.experimental import pallas as pl
from jax.experimental.pallas import tpu as pltpu
from jax.experimental.pallas import tpu_sc as plsc

N = 10000
E = 320000
D = 128
H = 128
B = 64

NC, NS = 2, 16
NW = NC * NS
EPW = E // NW
ECHUNK = 50
NECHUNK = EPW // ECHUNK
NPH = 5
CPP = NECHUNK // NPH
RING = 4
RCHUNK = 40
NRCHUNK = N // RCHUNK
RCPT = -(-NRCHUNK // NS)


def _sc_mesh():
    return plsc.VectorSubcoreMesh(core_axis_name="c", subcore_axis_name="s")


def _deg_body(dst_hbm, out_hbm, di_v, ones_v, zb_v, hist_sh):
    c = lax.axis_index("c")
    s = lax.axis_index("s")
    w = c * NS + s

    def fill(i, _):
        ones_v[i, :] = jnp.ones((16,), jnp.float32)
        return 0

    lax.fori_loop(0, ECHUNK, fill, 0)

    def zfill(i, _):
        zb_v[i, :] = jnp.zeros((16,), jnp.float32)
        return 0

    lax.fori_loop(0, RCHUNK, zfill, 0)
    pltpu.sync_copy(dst_hbm.at[w], di_v)

    def zstep(j, _):
        rc = s + NS * j

        @pl.when(rc < NRCHUNK)
        def _():
            pltpu.sync_copy(zb_v, hist_sh.at[pl.ds(rc * RCHUNK, RCHUNK)])
        return 0

    lax.fori_loop(0, RCPT, zstep, 0)
    plsc.subcore_barrier()

    def step(t, _):
        pltpu.sync_copy(ones_v, hist_sh.at[di_v.at[t]], add=True)
        return 0

    lax.fori_loop(0, NECHUNK, step, 0)
    plsc.subcore_barrier()

    def wstep(j, _):
        rc = s + NS * j

        @pl.when(rc < NRCHUNK)
        def _():
            pltpu.sync_copy(hist_sh.at[pl.ds(rc * RCHUNK, RCHUNK)],
                            out_hbm.at[c, pl.ds(rc * RCHUNK, RCHUNK)])
        return 0

    lax.fori_loop(0, RCPT, wstep, 0)


def _deg(dst3d):
    f = pl.kernel(
        _deg_body,
        out_type=jax.ShapeDtypeStruct((NC, N, 16), jnp.float32),
        mesh=_sc_mesh(),
        scratch_types=[
            pltpu.VMEM((NECHUNK, ECHUNK), jnp.int32),
            pltpu.VMEM((ECHUNK, 16), jnp.float32),
            pltpu.VMEM((RCHUNK, 16), jnp.float32),
            pltpu.VMEM_SHARED((N, 16), jnp.float32),
        ],
    )
    return f(dst3d)


def _agg_body(src_hbm, dst_hbm, y_hbm, out_hbm, si_v, di_v, rows,
              acc_sh, gsems):
    c = lax.axis_index("c")
    s = lax.axis_index("s")
    w = c * NS + s

    def zfill(i, _):
        for k in range(H // 16):
            rows[0][i, pl.ds(k * 16, 16)] = jnp.zeros((16,), jnp.float32)
        return 0

    lax.fori_loop(0, RCHUNK, zfill, 0)

    def zstep(j, _):
        rc = s + NS * j

        @pl.when(rc < NRCHUNK)
        def _():
            pltpu.sync_copy(rows[0].at[pl.ds(0, RCHUNK)],
                            acc_sh.at[pl.ds(rc * RCHUNK, RCHUNK)])
        return 0

    lax.fori_loop(0, RCPT, zstep, 0)
    plsc.subcore_barrier()

    for p in range(NPH):
        pltpu.sync_copy(src_hbm.at[w, p], si_v)
        pltpu.sync_copy(dst_hbm.at[w, p], di_v)
        for k in range(RING):
            pltpu.async_copy(y_hbm.at[si_v.at[k]], rows[k], gsems[k])

        def step(t, _):
            j0 = RING * t
            for k in range(RING):
                pltpu.make_async_copy(y_hbm.at[si_v.at[j0 + k]], rows[k],
                                      gsems[k]).wait()
                pltpu.sync_copy(rows[k], acc_sh.at[di_v.at[j0 + k]], add=True)

                @pl.when(t < CPP // RING - 1)
                def _():
                    pltpu.async_copy(y_hbm.at[si_v.at[j0 + RING + k]],
                                     rows[k], gsems[k])
            return 0

        lax.fori_loop(0, CPP // RING, step, 0)
    plsc.subcore_barrier()

    def wstep(j, _):
        rc = s + NS * j

        @pl.when(rc < NRCHUNK)
        def _():
            pltpu.sync_copy(acc_sh.at[pl.ds(rc * RCHUNK, RCHUNK)],
                            out_hbm.at[c, pl.ds(rc * RCHUNK, RCHUNK)])
        return 0

    lax.fori_loop(0, RCPT, wstep, 0)


def _agg_body_flat(src_hbm, dst_hbm, y_hbm, out_hbm, si_v, di_v,
                   r0, r1, r2, r3, acc_sh, g0, g1, g2, g3):
    return _agg_body(src_hbm, dst_hbm, y_hbm, out_hbm, si_v, di_v,
                     [r0, r1, r2, r3], acc_sh, [g0, g1, g2, g3])


def _agg(src3d, dst3d, y):
    f = pl.kernel(
        _agg_body_flat,
        out_type=jax.ShapeDtypeStruct((NC, N, H), jnp.float32),
        mesh=_sc_mesh(),
        scratch_types=(
            [pltpu.VMEM((CPP, ECHUNK), jnp.int32)] * 2
            + [pltpu.VMEM((ECHUNK, H), jnp.float32)] * RING
            + [pltpu.VMEM_SHARED((N, H), jnp.float32)]
            + [pltpu.SemaphoreType.DMA] * RING
        ),
    )
    return f(src3d, dst3d, y)


R = 1000
GRID = N // R


def _dinv(hp):
    deg = hp[0, :, 0] + hp[1, :, 0] + 1.0
    return lax.rsqrt(deg)


def _mm_body(x_ref, w_ref, y_ref):
    y_ref[...] = jnp.dot(x_ref[...], w_ref[...],
                         preferred_element_type=jnp.float32)


def _mm(x, w):
    return pl.pallas_call(
        _mm_body,
        grid=(GRID,),
        in_specs=[
            pl.BlockSpec((R, D), lambda i: (i, 0)),
            pl.BlockSpec((D, H), lambda i: (0, 0)),
        ],
        out_specs=pl.BlockSpec((R, H), lambda i: (i, 0)),
        out_shape=jax.ShapeDtypeStruct((N, H), jnp.float32),
    )(x, w)


def _scale_body(hp_ref, xw_ref, y_ref):
    dinv = _dinv(hp_ref[...])
    y_ref[...] = dinv[:, None] * xw_ref[...]


def _scale(hp, xw):
    return pl.pallas_call(
        _scale_body,
        grid=(GRID,),
        in_specs=[
            pl.BlockSpec((NC, R, 16), lambda i: (0, i, 0)),
            pl.BlockSpec((R, H), lambda i: (i, 0)),
        ],
        out_specs=pl.BlockSpec((R, H), lambda i: (i, 0)),
        out_shape=jax.ShapeDtypeStruct((N, H), jnp.float32),
    )(hp, xw)


def _layer_body(hp_ref, ap_ref, y_ref, w_ref, b_ref, o_ref):
    dinv = _dinv(hp_ref[...])
    aa = ap_ref[...]
    a = aa[0] + aa[1] + y_ref[...]
    h = jnp.maximum(dinv[:, None] * a + b_ref[...], 0.0)
    o_ref[...] = dinv[:, None] * jnp.dot(h, w_ref[...],
                                         preferred_element_type=jnp.float32)


def _layer(hp, ap, y, w, b):
    return pl.pallas_call(
        _layer_body,
        grid=(GRID,),
        in_specs=[
            pl.BlockSpec((NC, R, 16), lambda i: (0, i, 0)),
            pl.BlockSpec((NC, R, H), lambda i: (0, i, 0)),
            pl.BlockSpec((R, H), lambda i: (i, 0)),
            pl.BlockSpec((H, H), lambda i: (0, 0)),
            pl.BlockSpec((1, H), lambda i: (0, 0)),
        ],
        out_specs=pl.BlockSpec((R, H), lambda i: (i, 0)),
        out_shape=jax.ShapeDtypeStruct((N, H), jnp.float32),
    )(hp, ap, y, w, b)


def _final_body(hp_ref, ap_ref, y_ref, b2_ref, batch_ref, l1w_ref, l1b_ref,
                l2w_ref, l2b_ref, o_ref, pooled_acc, cnt_acc):
    i = pl.program_id(0)
    dinv = _dinv(hp_ref[...])
    aa = ap_ref[...]
    a = aa[0] + aa[1] + y_ref[...]
    h = jnp.maximum(dinv[:, None] * a + b2_ref[...], 0.0)

    bid = batch_ref[...]
    lanes = lax.broadcasted_iota(jnp.int32, (R, B), 1)
    m = jnp.where(bid == lanes, 1.0, 0.0)
    p = lax.dot_general(m, h, (((0,), (0,)), ((), ())),
                        preferred_element_type=jnp.float32)
    cn = lax.dot_general(m, jnp.ones_like(h), (((0,), (0,)), ((), ())),
                         preferred_element_type=jnp.float32)

    @pl.when(i == 0)
    def _():
        pooled_acc[...] = jnp.zeros_like(pooled_acc)
        cnt_acc[...] = jnp.zeros_like(cnt_acc)

    pooled_acc[...] += p
    cnt_acc[...] += cn

    @pl.when(i == GRID - 1)
    def _():
        mean = pooled_acc[...] / jnp.maximum(cnt_acc[...], 1.0)
        z = jnp.maximum(jnp.dot(mean, l1w_ref[...],
                                preferred_element_type=jnp.float32)
                        + l1b_ref[...], 0.0)
        o_ref[...] = (jnp.dot(z, l2w_ref[...],
                              preferred_element_type=jnp.float32)
                      + l2b_ref[...])


def _final(hp, ap, y, b2, batch2d, l1w, l1b, l2w, l2b):
    return pl.pallas_call(
        _final_body,
        grid=(GRID,),
        in_specs=[
            pl.BlockSpec((NC, R, 16), lambda i: (0, i, 0)),
            pl.BlockSpec((NC, R, H), lambda i: (0, i, 0)),
            pl.BlockSpec((R, H), lambda i: (i, 0)),
            pl.BlockSpec((1, H), lambda i: (0, 0)),
            pl.BlockSpec((R, 1), lambda i: (i, 0)),
            pl.BlockSpec((H, B), lambda i: (0, 0)),
            pl.BlockSpec((1, B), lambda i: (0, 0)),
            pl.BlockSpec((B, 1), lambda i: (0, 0)),
            pl.BlockSpec((1, 1), lambda i: (0, 0)),
        ],
        out_specs=pl.BlockSpec((B, 1), lambda i: (0, 0)),
        out_shape=jax.ShapeDtypeStruct((B, 1), jnp.float32),
        scratch_shapes=[
            pltpu.VMEM((B, H), jnp.float32),
            pltpu.VMEM((B, H), jnp.float32),
        ],
        compiler_params=pltpu.CompilerParams(
            dimension_semantics=("arbitrary",)),
    )(hp, ap, y, b2, batch2d, l1w, l1b, l2w, l2b)


def kernel(x, edge_index, batch, W1, b1, W2, b2, l1W, l1b, l2W, l2b):
    src = edge_index[0].astype(jnp.int32).reshape(NW, NPH, CPP, ECHUNK)
    dst = edge_index[1].astype(jnp.int32).reshape(NW, NPH, CPP, ECHUNK)
    xw1 = _mm(x, W1)
    hp = _deg(dst.reshape(NW, NECHUNK, ECHUNK))
    y1 = _scale(hp, xw1)
    ap1 = _agg(src, dst, y1)
    y2 = _layer(hp, ap1, y1, W2, b1.reshape(1, H))
    ap2 = _agg(src, dst, y2)
    out = _final(hp, ap2, y2, b2.reshape(1, H), batch.astype(jnp.int32).reshape(N, 1),
                 l1W, l1b.reshape(1, B), l2W, l2b.reshape(1, 1))
    return out[:, 0]


# speedup vs baseline: 1.0934x; 1.0345x over previous
import functools

import jax
import jax.numpy as jnp
from jax import lax
from jax.experimental import pallas as pl
from jax.experimental.pallas import tpu as pltpu
from jax.experimental.pallas import tpu_sc as plsc

N = 10000
E = 320000
D = 128
H = 128
B = 64

NC, NS = 2, 16
NW = NC * NS
EPW = E // NW
ECHUNK = 50
NECHUNK = EPW // ECHUNK
NPH = 5
CPP = NECHUNK // NPH
RING = 4
RCHUNK = 40
NRCHUNK = N // RCHUNK
RCPT = -(-NRCHUNK // NS)


def _sc_mesh():
    return plsc.VectorSubcoreMesh(core_axis_name="c", subcore_axis_name="s")


def _deg_body(dst_hbm, out_hbm, di_v, ones_v, zb_v, hist_sh, hsem):
    c = lax.axis_index("c")
    s = lax.axis_index("s")
    w = c * NS + s

    def fill(i, _):
        ones_v[i, :] = jnp.ones((16,), jnp.float32)
        return 0

    lax.fori_loop(0, ECHUNK, fill, 0)

    def zfill(i, _):
        zb_v[i, :] = jnp.zeros((16,), jnp.float32)
        return 0

    lax.fori_loop(0, RCHUNK, zfill, 0)
    pltpu.sync_copy(dst_hbm.at[w], di_v)

    def zstep(j, _):
        rc = s + NS * j

        @pl.when(rc < NRCHUNK)
        def _():
            pltpu.sync_copy(zb_v, hist_sh.at[pl.ds(rc * RCHUNK, RCHUNK)])
        return 0

    lax.fori_loop(0, RCPT, zstep, 0)
    plsc.subcore_barrier()

    def step(t, _):
        pltpu.async_copy(ones_v, hist_sh.at[di_v.at[t]], hsem, add=True)
        return 0

    lax.fori_loop(0, NECHUNK, step, 0)

    def drain(t, _):
        pltpu.make_async_copy(ones_v, hist_sh.at[di_v.at[t]], hsem).wait()
        return 0

    lax.fori_loop(0, NECHUNK, drain, 0)
    plsc.subcore_barrier()

    def wstep(j, _):
        rc = s + NS * j

        @pl.when(rc < NRCHUNK)
        def _():
            pltpu.sync_copy(hist_sh.at[pl.ds(rc * RCHUNK, RCHUNK)],
                            out_hbm.at[c, pl.ds(rc * RCHUNK, RCHUNK)])
        return 0

    lax.fori_loop(0, RCPT, wstep, 0)


def _deg(dst3d):
    f = pl.kernel(
        _deg_body,
        out_type=jax.ShapeDtypeStruct((NC, N, 16), jnp.float32),
        mesh=_sc_mesh(),
        scratch_types=[
            pltpu.VMEM((NECHUNK, ECHUNK), jnp.int32),
            pltpu.VMEM((ECHUNK, 16), jnp.float32),
            pltpu.VMEM((RCHUNK, 16), jnp.float32),
            pltpu.VMEM_SHARED((N, 16), jnp.float32),
            pltpu.SemaphoreType.DMA,
        ],
    )
    return f(dst3d)


def _agg_body(src_hbm, dst_hbm, y_hbm, out_hbm, si_v, di_v, rows,
              acc_sh, gsems):
    c = lax.axis_index("c")
    s = lax.axis_index("s")
    w = c * NS + s

    def zfill(i, _):
        for k in range(H // 16):
            rows[0][i, pl.ds(k * 16, 16)] = jnp.zeros((16,), jnp.float32)
        return 0

    lax.fori_loop(0, RCHUNK, zfill, 0)

    def zstep(j, _):
        rc = s + NS * j

        @pl.when(rc < NRCHUNK)
        def _():
            pltpu.sync_copy(rows[0].at[pl.ds(0, RCHUNK)],
                            acc_sh.at[pl.ds(rc * RCHUNK, RCHUNK)])
        return 0

    lax.fori_loop(0, RCPT, zstep, 0)
    plsc.subcore_barrier()

    for p in range(NPH):
        pltpu.sync_copy(src_hbm.at[w, p], si_v)
        pltpu.sync_copy(dst_hbm.at[w, p], di_v)
        for k in range(RING):
            pltpu.async_copy(y_hbm.at[si_v.at[k]], rows[k], gsems[k])

        def step(t, _):
            j0 = RING * t
            for k in range(RING):
                pltpu.make_async_copy(y_hbm.at[si_v.at[j0 + k]], rows[k],
                                      gsems[k]).wait()
                pltpu.sync_copy(rows[k], acc_sh.at[di_v.at[j0 + k]], add=True)

                @pl.when(t < CPP // RING - 1)
                def _():
                    pltpu.async_copy(y_hbm.at[si_v.at[j0 + RING + k]],
                                     rows[k], gsems[k])
            return 0

        lax.fori_loop(0, CPP // RING, step, 0)
    plsc.subcore_barrier()

    def wstep(j, _):
        rc = s + NS * j

        @pl.when(rc < NRCHUNK)
        def _():
            pltpu.sync_copy(acc_sh.at[pl.ds(rc * RCHUNK, RCHUNK)],
                            out_hbm.at[c, pl.ds(rc * RCHUNK, RCHUNK)])
        return 0

    lax.fori_loop(0, RCPT, wstep, 0)


def _agg_body_flat(src_hbm, dst_hbm, y_hbm, out_hbm, si_v, di_v,
                   r0, r1, r2, r3, acc_sh, g0, g1, g2, g3):
    return _agg_body(src_hbm, dst_hbm, y_hbm, out_hbm, si_v, di_v,
                     [r0, r1, r2, r3], acc_sh, [g0, g1, g2, g3])


def _agg(src3d, dst3d, y):
    f = pl.kernel(
        _agg_body_flat,
        out_type=jax.ShapeDtypeStruct((NC, N, H), jnp.float32),
        mesh=_sc_mesh(),
        scratch_types=(
            [pltpu.VMEM((CPP, ECHUNK), jnp.int32)] * 2
            + [pltpu.VMEM((ECHUNK, H), jnp.float32)] * RING
            + [pltpu.VMEM_SHARED((N, H), jnp.float32)]
            + [pltpu.SemaphoreType.DMA] * RING
        ),
    )
    return f(src3d, dst3d, y)


R = 1000
GRID = N // R


def _dinv(hp):
    deg = hp[0, :, 0] + hp[1, :, 0] + 1.0
    return lax.rsqrt(deg)


def _mm_body(x_ref, w_ref, y_ref):
    y_ref[...] = jnp.dot(x_ref[...], w_ref[...],
                         preferred_element_type=jnp.float32)


def _mm(x, w):
    return pl.pallas_call(
        _mm_body,
        grid=(GRID,),
        in_specs=[
            pl.BlockSpec((R, D), lambda i: (i, 0)),
            pl.BlockSpec((D, H), lambda i: (0, 0)),
        ],
        out_specs=pl.BlockSpec((R, H), lambda i: (i, 0)),
        out_shape=jax.ShapeDtypeStruct((N, H), jnp.float32),
    )(x, w)


def _scale_body(hp_ref, xw_ref, y_ref):
    dinv = _dinv(hp_ref[...])
    y_ref[...] = dinv[:, None] * xw_ref[...]


def _scale(hp, xw):
    return pl.pallas_call(
        _scale_body,
        grid=(GRID,),
        in_specs=[
            pl.BlockSpec((NC, R, 16), lambda i: (0, i, 0)),
            pl.BlockSpec((R, H), lambda i: (i, 0)),
        ],
        out_specs=pl.BlockSpec((R, H), lambda i: (i, 0)),
        out_shape=jax.ShapeDtypeStruct((N, H), jnp.float32),
    )(hp, xw)


def _layer_body(hp_ref, ap_ref, y_ref, w_ref, b_ref, o_ref):
    dinv = _dinv(hp_ref[...])
    aa = ap_ref[...]
    a = aa[0] + aa[1] + y_ref[...]
    h = jnp.maximum(dinv[:, None] * a + b_ref[...], 0.0)
    o_ref[...] = dinv[:, None] * jnp.dot(h, w_ref[...],
                                         preferred_element_type=jnp.float32)


def _layer(hp, ap, y, w, b):
    return pl.pallas_call(
        _layer_body,
        grid=(GRID,),
        in_specs=[
            pl.BlockSpec((NC, R, 16), lambda i: (0, i, 0)),
            pl.BlockSpec((NC, R, H), lambda i: (0, i, 0)),
            pl.BlockSpec((R, H), lambda i: (i, 0)),
            pl.BlockSpec((H, H), lambda i: (0, 0)),
            pl.BlockSpec((1, H), lambda i: (0, 0)),
        ],
        out_specs=pl.BlockSpec((R, H), lambda i: (i, 0)),
        out_shape=jax.ShapeDtypeStruct((N, H), jnp.float32),
    )(hp, ap, y, w, b)


def _final_body(hp_ref, ap_ref, y_ref, b2_ref, batch_ref, l1w_ref, l1b_ref,
                l2w_ref, l2b_ref, o_ref, pooled_acc, cnt_acc):
    i = pl.program_id(0)
    dinv = _dinv(hp_ref[...])
    aa = ap_ref[...]
    a = aa[0] + aa[1] + y_ref[...]
    h = jnp.maximum(dinv[:, None] * a + b2_ref[...], 0.0)

    bid = batch_ref[...]
    lanes = lax.broadcasted_iota(jnp.int32, (R, B), 1)
    m = jnp.where(bid == lanes, 1.0, 0.0)
    p = lax.dot_general(m, h, (((0,), (0,)), ((), ())),
                        preferred_element_type=jnp.float32)
    cn = lax.dot_general(m, jnp.ones_like(h), (((0,), (0,)), ((), ())),
                         preferred_element_type=jnp.float32)

    @pl.when(i == 0)
    def _():
        pooled_acc[...] = jnp.zeros_like(pooled_acc)
        cnt_acc[...] = jnp.zeros_like(cnt_acc)

    pooled_acc[...] += p
    cnt_acc[...] += cn

    @pl.when(i == GRID - 1)
    def _():
        mean = pooled_acc[...] / jnp.maximum(cnt_acc[...], 1.0)
        z = jnp.maximum(jnp.dot(mean, l1w_ref[...],
                                preferred_element_type=jnp.float32)
                        + l1b_ref[...], 0.0)
        o_ref[...] = (jnp.dot(z, l2w_ref[...],
                              preferred_element_type=jnp.float32)
                      + l2b_ref[...])


def _final(hp, ap, y, b2, batch2d, l1w, l1b, l2w, l2b):
    return pl.pallas_call(
        _final_body,
        grid=(GRID,),
        in_specs=[
            pl.BlockSpec((NC, R, 16), lambda i: (0, i, 0)),
            pl.BlockSpec((NC, R, H), lambda i: (0, i, 0)),
            pl.BlockSpec((R, H), lambda i: (i, 0)),
            pl.BlockSpec((1, H), lambda i: (0, 0)),
            pl.BlockSpec((R, 1), lambda i: (i, 0)),
            pl.BlockSpec((H, B), lambda i: (0, 0)),
            pl.BlockSpec((1, B), lambda i: (0, 0)),
            pl.BlockSpec((B, 1), lambda i: (0, 0)),
            pl.BlockSpec((1, 1), lambda i: (0, 0)),
        ],
        out_specs=pl.BlockSpec((B, 1), lambda i: (0, 0)),
        out_shape=jax.ShapeDtypeStruct((B, 1), jnp.float32),
        scratch_shapes=[
            pltpu.VMEM((B, H), jnp.float32),
            pltpu.VMEM((B, H), jnp.float32),
        ],
        compiler_params=pltpu.CompilerParams(
            dimension_semantics=("arbitrary",)),
    )(hp, ap, y, b2, batch2d, l1w, l1b, l2w, l2b)


def kernel(x, edge_index, batch, W1, b1, W2, b2, l1W, l1b, l2W, l2b):
    src = edge_index[0].astype(jnp.int32).reshape(NW, NPH, CPP, ECHUNK)
    dst = edge_index[1].astype(jnp.int32).reshape(NW, NPH, CPP, ECHUNK)
    xw1 = _mm(x, W1)
    hp = _deg(dst.reshape(NW, NECHUNK, ECHUNK))
    y1 = _scale(hp, xw1)
    ap1 = _agg(src, dst, y1)
    y2 = _layer(hp, ap1, y1, W2, b1.reshape(1, H))
    ap2 = _agg(src, dst, y2)
    out = _final(hp, ap2, y2, b2.reshape(1, H), batch.astype(jnp.int32).reshape(N, 1),
                 l1W, l1b.reshape(1, B), l2W, l2b.reshape(1, 1))
    return out[:, 0]
